# Initial kernel scaffold; baseline (speedup 1.0000x reference)
import jax
import jax.numpy as jnp
from jax.experimental import pallas as pl


def kernel(x, edge_index, edges, W1l, b1, W1r, W2l, b2, W2r):
    raise NotImplementedError("write your pallas kernel here")


# speedup vs baseline: 3.6017x; 3.6017x over previous
import functools

import jax
import jax.numpy as jnp
from jax import lax
from jax.experimental import pallas as pl
from jax.experimental.pallas import tpu as pltpu
from jax.experimental.pallas import tpu_sc as plsc

N_NODES = 10000
D = 128
NC = 2
NS = 16
NW = NC * NS
CHUNK = 128
IGRP = 8
DP = D + 16
SRPT = 80
RPT = 640
N_PAD = NS * RPT

_mesh = plsc.VectorSubcoreMesh(core_axis_name="c", subcore_axis_name="s")


def _mm2_body(x_ref, wl_ref, wr_ref, b_ref, y_ref, r_ref):
    xb = x_ref[...]
    dn = (((1,), (1,)), ((), ()))
    y_ref[...] = lax.dot_general(xb, wl_ref[...], dn,
                                 preferred_element_type=jnp.float32)
    r_ref[...] = lax.dot_general(xb, wr_ref[...], dn,
                                 preferred_element_type=jnp.float32) + b_ref[...]


def _tc_premul(x, wl, wr, b):
    return pl.pallas_call(
        _mm2_body,
        out_shape=(jax.ShapeDtypeStruct(x.shape, jnp.float32),
                   jax.ShapeDtypeStruct(x.shape, jnp.float32)),
    )(x, wl, wr, b.reshape(1, D))


def _combine_body(acc_ref, r_ref, out_ref, *, relu):
    agg = acc_ref[0, :, :D] + acc_ref[1, :, :D]
    deg = acc_ref[0, :, D:D + 1] + acc_ref[1, :, D:D + 1]
    h = agg / jnp.maximum(deg, 1.0) + r_ref[...]
    out_ref[...] = jnp.maximum(h, 0.0) if relu else h


def _tc_combine(acc, r, relu):
    return pl.pallas_call(
        functools.partial(_combine_body, relu=relu),
        out_shape=jax.ShapeDtypeStruct(r.shape, jnp.float32),
    )(acc, r)


def _seg_body(n_chunks, y_hbm, src_hbm, dst_hbm, zacc_hbm, acc_out,
              src_v, dst_v, rows_v, stage_v, acc_sh):
    c = lax.axis_index("c")
    s = lax.axis_index("s")
    wid = c * NS + s
    row0 = s * RPT

    for t in range(RPT // SRPT):
        r = row0 + t * SRPT
        pltpu.sync_copy(zacc_hbm.at[pl.ds(r, SRPT)], stage_v)
        pltpu.sync_copy(stage_v, acc_sh.at[pl.ds(r, SRPT)])
    plsc.subcore_barrier()

    n_groups = n_chunks // IGRP

    def group(g, carry):
        pltpu.sync_copy(src_hbm.at[wid, pl.ds(g * IGRP, IGRP)], src_v)
        pltpu.sync_copy(dst_hbm.at[wid, pl.ds(g * IGRP, IGRP)], dst_v)
        for i in range(IGRP):
            pltpu.sync_copy(y_hbm.at[src_v.at[i]], rows_v)
            pltpu.sync_copy(rows_v, acc_sh.at[dst_v.at[i]], add=True)
        return carry

    lax.fori_loop(0, n_groups, group, 0)
    plsc.subcore_barrier()

    for t in range(RPT // SRPT):
        r = row0 + t * SRPT
        pltpu.sync_copy(acc_sh.at[pl.ds(r, SRPT)], stage_v)
        pltpu.sync_copy(stage_v, acc_out.at[c, pl.ds(r, SRPT)])


def _sc_segsum(y, src3, dst3, zacc):
    n_chunks = src3.shape[1]
    fn = pl.kernel(
        functools.partial(_seg_body, n_chunks),
        out_type=jax.ShapeDtypeStruct((NC, N_PAD, DP), jnp.float32),
        mesh=_mesh,
        compiler_params=pltpu.CompilerParams(use_tc_tiling_on_sc=False),
        scratch_types=(
            pltpu.VMEM((IGRP, CHUNK), jnp.int32),
            pltpu.VMEM((IGRP, CHUNK), jnp.int32),
            pltpu.VMEM((CHUNK, DP), jnp.float32),
            pltpu.VMEM((SRPT, DP), jnp.float32),
            pltpu.VMEM_SHARED((N_PAD, DP), jnp.float32),
        ),
    )
    return fn(y, src3, dst3, zacc)


def _dec_body(n_chunks, z_hbm, es_hbm, ed_hbm, out_hbm,
              es_v, ed_v, zs_v, zd_v, out_v):
    c = lax.axis_index("c")
    s = lax.axis_index("s")
    wid = c * NS + s
    pltpu.sync_copy(es_hbm.at[wid], es_v)
    pltpu.sync_copy(ed_hbm.at[wid], ed_v)
    lane = lax.iota(jnp.int32, 16)

    _gdn = lax.GatherDimensionNumbers(
        offset_dims=(), collapsed_slice_dims=(0,), start_index_map=(0,))

    def _perm(a, idx):
        return lax.gather(a, idx[:, None], _gdn, slice_sizes=(1,),
                          mode=lax.GatherScatterMode.PROMISE_IN_BOUNDS)

    def _combine(a, b, bit):
        m = (lane & bit) == 0
        pa = _perm(a, lane ^ bit)
        pb = _perm(b, lane ^ bit)
        return jnp.where(m, a, pb) + jnp.where(m, pa, b)

    def chunk(i, carry):
        pltpu.sync_copy(z_hbm.at[es_v.at[i]], zs_v)
        pltpu.sync_copy(z_hbm.at[ed_v.at[i]], zd_v)

        def group(g, carry2):
            vs = []
            for j in range(16):
                row = g * 16 + j
                acc = jnp.zeros((16,), jnp.float32)
                for k in range(D // 16):
                    acc = acc + (zs_v[row, pl.ds(k * 16, 16)]
                                 * zd_v[row, pl.ds(k * 16, 16)])
                vs.append(acc)
            for bit in (1, 2, 4, 8):
                vs = [_combine(vs[2 * t], vs[2 * t + 1], bit)
                      for t in range(len(vs) // 2)]
            out_v[i, pl.ds(g * 16, 16)] = vs[0]
            return carry2

        lax.fori_loop(0, CHUNK // 16, group, 0)
        return carry

    lax.fori_loop(0, n_chunks, chunk, 0)
    pltpu.sync_copy(out_v, out_hbm.at[wid])


def _sc_decode(z, es3, ed3):
    n_chunks = es3.shape[1]
    fn = pl.kernel(
        functools.partial(_dec_body, n_chunks),
        out_type=jax.ShapeDtypeStruct((NW, n_chunks, CHUNK), jnp.float32),
        mesh=_mesh,
        scratch_types=(
            pltpu.VMEM((n_chunks, CHUNK), jnp.int32),
            pltpu.VMEM((n_chunks, CHUNK), jnp.int32),
            pltpu.VMEM((CHUNK, D), jnp.float32),
            pltpu.VMEM((CHUNK, D), jnp.float32),
            pltpu.VMEM((n_chunks, CHUNK), jnp.float32),
        ),
    )
    return fn(z, es3, ed3)


def _pad_idx(idx, fill):
    e = idx.shape[0]
    e_pad = -(-e // (NW * CHUNK)) * (NW * CHUNK)
    n_chunks = e_pad // (NW * CHUNK)
    out = jnp.full((e_pad,), fill, jnp.int32).at[:e].set(idx.astype(jnp.int32))
    return out.reshape(NW, n_chunks, CHUNK)


def kernel(x, edge_index, edges, W1l, b1, W1r, W2l, b2, W2r):
    n, d = x.shape
    e = edge_index.shape[1]
    x_pad = jnp.zeros((N_PAD, d), jnp.float32).at[:n].set(x)

    src3 = _pad_idx(edge_index[0], 0)
    dst3 = _pad_idx(edge_index[1], N_NODES)
    es3 = _pad_idx(edges[:, 0], 0)
    ed3 = _pad_idx(edges[:, 1], 0)

    zacc = jnp.zeros((N_PAD, DP), jnp.float32)
    onecol = jnp.ones((N_PAD, DP - D), jnp.float32)

    y1, r1 = _tc_premul(x_pad, W1l, W1r, b1)
    acc1 = _sc_segsum(jnp.concatenate([y1, onecol], axis=1), src3, dst3, zacc)
    h = _tc_combine(acc1, r1, relu=True)

    y2, r2 = _tc_premul(h, W2l, W2r, b2)
    acc2 = _sc_segsum(jnp.concatenate([y2, onecol], axis=1), src3, dst3, zacc)
    z = _tc_combine(acc2, r2, relu=False)

    out3 = _sc_decode(z, es3, ed3)
    return out3.reshape(-1)[:e]
